# baseline (speedup 1.0000x reference)
import functools

import jax
import jax.numpy as jnp
from jax import lax
from jax.experimental import pallas as pl
from jax.experimental.pallas import tpu as pltpu
from jax.experimental.pallas import tpu_sc as plsc

NC, NS = 2, 16
NW = NC * NS
B = 4096
D = 32
BW = B // NW
H = 200
HC = 100
W5 = 50
T = 5
F32 = jnp.float32


def _acc_rows(buf_ref, n, pre):
    accs = [jnp.zeros((16,), F32) for _ in range(8)]
    for t in range(n):
        c = (t % 4) * 2
        accs[c] = accs[c] + buf_ref[pre + (t, pl.ds(0, 16))]
        accs[c + 1] = accs[c + 1] + buf_ref[pre + (t, pl.ds(16, 16))]
    lo = (accs[0] + accs[2]) + (accs[4] + accs[6])
    hi = (accs[1] + accs[3]) + (accs[5] + accs[7])
    return lo, hi


def _sc_pool(hist3, wish, tags_t, bid, auth, lang,
             book_emb, auth_emb, lang_emb, tag_emb):
    mesh = plsc.VectorSubcoreMesh(core_axis_name="c", subcore_axis_name="s",
                                  num_cores=NC, num_subcores=NS)

    @functools.partial(
        pl.kernel,
        out_type=(jax.ShapeDtypeStruct((B, D), F32),
                  jax.ShapeDtypeStruct((B, D), F32)),
        mesh=mesh,
        scratch_types=[
            pltpu.VMEM((BW, 2, HC), jnp.int32),
            pltpu.VMEM((BW, W5), jnp.int32),
            pltpu.VMEM((T, BW), jnp.int32),
            pltpu.VMEM((BW,), jnp.int32),
            pltpu.VMEM((BW,), jnp.int32),
            pltpu.VMEM((BW,), jnp.int32),
            pltpu.VMEM((4, H, D), F32),
            pltpu.VMEM((4, W5, D), F32),
            pltpu.VMEM((BW, D), F32),
            pltpu.VMEM((BW, D), F32),
            pltpu.VMEM((BW, D), F32),
            pltpu.VMEM((T, BW, D), F32),
            pltpu.VMEM((BW, D), F32),
            pltpu.VMEM((BW, D), F32),
            pltpu.SemaphoreType.DMA,
            pltpu.SemaphoreType.DMA,
            pltpu.SemaphoreType.DMA,
            pltpu.SemaphoreType.DMA,
            pltpu.SemaphoreType.DMA,
        ],
        compiler_params=pltpu.CompilerParams(use_tc_tiling_on_sc=False),
    )
    def k(hist_hbm, wish_hbm, tagt_hbm, bid_hbm, auth_hbm, lang_hbm,
          book_hbm, authe_hbm, lange_hbm, tage_hbm,
          u_hbm, item_hbm,
          hist_v, wish_v, tagt_v, bid_v, auth_v, lang_v,
          hbuf, wbuf, bidr, authr, langr, tagr, u_o, it_o,
          sem0, sem1, sem2, sem3, semi):
        wid = lax.axis_index("s") * NC + lax.axis_index("c")
        base = wid * BW
        sems = [sem0, sem1, sem2, sem3]
        NB = 4

        stage_cps = [
            pltpu.async_copy(hist_hbm.at[pl.ds(base, BW)], hist_v, semi),
            pltpu.async_copy(wish_hbm.at[pl.ds(base, BW)], wish_v, semi),
            pltpu.async_copy(tagt_hbm.at[:, pl.ds(base, BW)], tagt_v, semi),
            pltpu.async_copy(bid_hbm.at[pl.ds(base, BW)], bid_v, semi),
            pltpu.async_copy(auth_hbm.at[pl.ds(base, BW)], auth_v, semi),
            pltpu.async_copy(lang_hbm.at[pl.ds(base, BW)], lang_v, semi),
        ]
        for cp in stage_cps:
            cp.wait()

        item_cps = [
            pltpu.async_copy(book_hbm.at[bid_v], bidr, semi),
            pltpu.async_copy(authe_hbm.at[auth_v], authr, semi),
            pltpu.async_copy(lange_hbm.at[lang_v], langr, semi),
        ]
        for t in range(T):
            item_cps.append(
                pltpu.async_copy(tage_hbm.at[tagt_v.at[t]], tagr.at[t], semi))

        def issue(j, s):
            pltpu.async_copy(book_hbm.at[hist_v.at[j, 0]],
                             hbuf.at[s, pl.ds(0, HC)], sems[s])
            pltpu.async_copy(book_hbm.at[hist_v.at[j, 1]],
                             hbuf.at[s, pl.ds(HC, HC)], sems[s])
            pltpu.async_copy(book_hbm.at[wish_v.at[j]], wbuf.at[s], sems[s])

        for s in range(NB):
            issue(s, s)

        def row_group(g, _):
            for s in range(NB):
                j = g * NB + s
                pltpu.make_async_copy(book_hbm.at[pl.ds(0, HC)],
                                      hbuf.at[s, pl.ds(0, HC)], sems[s]).wait()
                pltpu.make_async_copy(book_hbm.at[pl.ds(0, HC)],
                                      hbuf.at[s, pl.ds(HC, HC)], sems[s]).wait()
                pltpu.make_async_copy(book_hbm.at[pl.ds(0, W5)],
                                      wbuf.at[s], sems[s]).wait()

                h_lo, h_hi = _acc_rows(hbuf, H, (s,))
                w_lo, w_hi = _acc_rows(wbuf, W5, (s,))
                u_o[j, pl.ds(0, 16)] = h_lo * (1.0 / H) + w_lo * (1.0 / W5)
                u_o[j, pl.ds(16, 16)] = h_hi * (1.0 / H) + w_hi * (1.0 / W5)

                jn = j + NB

                @pl.when(jn < BW)
                def _():
                    issue(jn, s)
            return 0

        lax.fori_loop(0, BW // NB, row_group, 0)

        for cp in item_cps:
            cp.wait()

        def item_body(j, _):
            for lo, sl in ((0, pl.ds(0, 16)), (16, pl.ds(16, 16))):
                s = bidr[j, sl] + authr[j, sl] + langr[j, sl]
                t01 = tagr[0, j, sl] + tagr[1, j, sl]
                t23 = tagr[2, j, sl] + tagr[3, j, sl]
                tsum = (t01 + t23) + tagr[4, j, sl]
                it_o[j, sl] = s + tsum * (1.0 / T)
            return 0

        lax.fori_loop(0, BW, item_body, 0)

        pltpu.sync_copy(u_o, u_hbm.at[pl.ds(base, BW)])
        pltpu.sync_copy(it_o, item_hbm.at[pl.ds(base, BW)])

    return k(hist3, wish, tags_t, bid, auth, lang,
             book_emb, auth_emb, lang_emb, tag_emb)


_CB = 2048


def _detile_body(in_ref, out_ref):
    x = in_ref[...]
    eye = jnp.eye(D, dtype=F32)
    w = [lax.dot_general(x[:, 512 * k:512 * (k + 1)], eye,
                         (((0,), (0,)), ((), ())),
                         preferred_element_type=F32)
         for k in range(4)]
    out_ref[...] = jnp.concatenate(w, axis=1).reshape(-1)


def _detile_table(tbl):
    V = tbl.shape[0]
    grid = (V + _CB - 1) // _CB
    out = pl.pallas_call(
        _detile_body,
        grid=(grid,),
        in_specs=[pl.BlockSpec((D, _CB), lambda c: (0, c))],
        out_specs=pl.BlockSpec((D * _CB,), lambda c: (c,)),
        out_shape=jax.ShapeDtypeStruct((grid * _CB * D,), F32),
    )(tbl.T)
    return out.reshape(grid * _CB, D)


def _psi(i):
    return (i & ~(_CB - 1)) | ((i & 511) << 2) | ((i & (_CB - 1)) >> 9)


def _tc_towers(u, item, dense8,
               dW1, db1, dW2, db2, dW3, db3,
               uW1, ub1, uW2, ub2, uW3, ub3):
    def body(u_ref, item_ref, dense_ref,
             dW1r, db1r, dW2r, db2r, dW3r, db3r,
             uW1r, ub1r, uW2r, ub2r, uW3r, ub3r, out_ref):
        def mm(x, w):
            return lax.dot_general(x, w, (((1,), (1,)), ((), ())),
                                   preferred_element_type=F32)

        x = jnp.maximum(mm(u_ref[:], uW1r[:]) + ub1r[:], 0.0)
        x = jnp.maximum(mm(x, uW2r[:]) + ub2r[:], 0.0)
        u_emb = mm(x, uW3r[:]) + ub3r[:]

        y = jnp.maximum(mm(dense_ref[:], dW1r[:]) + db1r[:], 0.0)
        y = jnp.maximum(mm(y, dW2r[:]) + db2r[:], 0.0)
        d_e = mm(y, dW3r[:]) + db3r[:]

        i_emb = item_ref[:] + d_e
        out_ref[:] = jnp.sum(u_emb * i_emb, axis=1, keepdims=True)

    return pl.pallas_call(
        body,
        out_shape=jax.ShapeDtypeStruct((B, 1), F32),
    )(u, item, dense8,
      dW1, db1, dW2, db2, dW3, db3,
      uW1, ub1, uW2, ub2, uW3, ub3)


def kernel(hist_ids, wish_ids, bid, auth, lang, tags, dense,
           book_emb, auth_emb, lang_emb, tag_emb,
           dW1, db1, dW2, db2, dW3, db3,
           uW1, ub1, uW2, ub2, uW3, ub3):
    hist3 = _psi(hist_ids).reshape(B, 2, HC)
    tags_t = _psi(tags).T
    u, item = _sc_pool(hist3, _psi(wish_ids), tags_t,
                       _psi(bid), _psi(auth), _psi(lang),
                       _detile_table(book_emb), _detile_table(auth_emb),
                       _detile_table(lang_emb), _detile_table(tag_emb))

    dense8 = jnp.concatenate(
        [dense, jnp.zeros((B, 5), F32)], axis=1)
    dW1p = jnp.concatenate([dW1, jnp.zeros((64, 5), F32)], axis=1)
    return _tc_towers(u, item, dense8,
                      dW1p, db1.reshape(1, 64), dW2, db2.reshape(1, 32),
                      dW3, db3.reshape(1, 32),
                      uW1, ub1.reshape(1, 64), uW2, ub2.reshape(1, 32),
                      uW3, ub3.reshape(1, 32))


# speedup vs baseline: 1.3326x; 1.3326x over previous
import functools

import jax
import jax.numpy as jnp
from jax import lax
from jax.experimental import pallas as pl
from jax.experimental.pallas import tpu as pltpu
from jax.experimental.pallas import tpu_sc as plsc

NC, NS = 2, 16
NW = NC * NS
B = 4096
D = 32
BW = B // NW
H = 200
HC = 100
W5 = 50
T = 5
F32 = jnp.float32


def _acc_rows(buf_ref, n, pre):
    accs = [jnp.zeros((16,), F32) for _ in range(8)]
    for t in range(n):
        c = (t % 4) * 2
        accs[c] = accs[c] + buf_ref[pre + (t, pl.ds(0, 16))]
        accs[c + 1] = accs[c + 1] + buf_ref[pre + (t, pl.ds(16, 16))]
    lo = (accs[0] + accs[2]) + (accs[4] + accs[6])
    hi = (accs[1] + accs[3]) + (accs[5] + accs[7])
    return lo, hi


def _sc_pool(hist3, wish, tags_t, bid, auth, lang,
             book_emb, auth_emb, lang_emb, tag_emb):
    mesh = plsc.VectorSubcoreMesh(core_axis_name="c", subcore_axis_name="s",
                                  num_cores=NC, num_subcores=NS)

    @functools.partial(
        pl.kernel,
        out_type=(jax.ShapeDtypeStruct((B, D), F32),
                  jax.ShapeDtypeStruct((B, D), F32)),
        mesh=mesh,
        scratch_types=[
            pltpu.VMEM((BW, 2, HC), jnp.int32),
            pltpu.VMEM((BW, W5), jnp.int32),
            pltpu.VMEM((T, BW), jnp.int32),
            pltpu.VMEM((BW,), jnp.int32),
            pltpu.VMEM((BW,), jnp.int32),
            pltpu.VMEM((BW,), jnp.int32),
            pltpu.VMEM((4, H, D), F32),
            pltpu.VMEM((4, W5, D), F32),
            pltpu.VMEM((BW, D), F32),
            pltpu.VMEM((BW, D), F32),
            pltpu.VMEM((BW, D), F32),
            pltpu.VMEM((T, BW, D), F32),
            pltpu.VMEM((BW, D), F32),
            pltpu.VMEM((BW, D), F32),
            pltpu.SemaphoreType.DMA,
            pltpu.SemaphoreType.DMA,
            pltpu.SemaphoreType.DMA,
            pltpu.SemaphoreType.DMA,
            pltpu.SemaphoreType.DMA,
        ],
        compiler_params=pltpu.CompilerParams(use_tc_tiling_on_sc=False),
    )
    def k(hist_hbm, wish_hbm, tagt_hbm, bid_hbm, auth_hbm, lang_hbm,
          book_hbm, authe_hbm, lange_hbm, tage_hbm,
          u_hbm, item_hbm,
          hist_v, wish_v, tagt_v, bid_v, auth_v, lang_v,
          hbuf, wbuf, bidr, authr, langr, tagr, u_o, it_o,
          sem0, sem1, sem2, sem3, semi):
        wid = lax.axis_index("s") * NC + lax.axis_index("c")
        base = wid * BW
        sems = [sem0, sem1, sem2, sem3]
        NB = 4

        stage_cps = [
            pltpu.async_copy(hist_hbm.at[pl.ds(base, BW)], hist_v, semi),
            pltpu.async_copy(wish_hbm.at[pl.ds(base, BW)], wish_v, semi),
            pltpu.async_copy(tagt_hbm.at[:, pl.ds(base, BW)], tagt_v, semi),
            pltpu.async_copy(bid_hbm.at[pl.ds(base, BW)], bid_v, semi),
            pltpu.async_copy(auth_hbm.at[pl.ds(base, BW)], auth_v, semi),
            pltpu.async_copy(lang_hbm.at[pl.ds(base, BW)], lang_v, semi),
        ]
        for cp in stage_cps:
            cp.wait()

        item_cps = [
            pltpu.async_copy(book_hbm.at[bid_v], bidr, semi),
            pltpu.async_copy(authe_hbm.at[auth_v], authr, semi),
            pltpu.async_copy(lange_hbm.at[lang_v], langr, semi),
        ]
        for t in range(T):
            item_cps.append(
                pltpu.async_copy(tage_hbm.at[tagt_v.at[t]], tagr.at[t], semi))

        def issue(j, s):
            pltpu.async_copy(book_hbm.at[hist_v.at[j, 0]],
                             hbuf.at[s, pl.ds(0, HC)], sems[s])
            pltpu.async_copy(book_hbm.at[hist_v.at[j, 1]],
                             hbuf.at[s, pl.ds(HC, HC)], sems[s])
            pltpu.async_copy(book_hbm.at[wish_v.at[j]], wbuf.at[s], sems[s])

        for s in range(NB):
            issue(s, s)

        def row_group(g, _):
            for s in range(NB):
                j = g * NB + s
                pltpu.make_async_copy(book_hbm.at[pl.ds(0, HC)],
                                      hbuf.at[s, pl.ds(0, HC)], sems[s]).wait()
                pltpu.make_async_copy(book_hbm.at[pl.ds(0, HC)],
                                      hbuf.at[s, pl.ds(HC, HC)], sems[s]).wait()
                pltpu.make_async_copy(book_hbm.at[pl.ds(0, W5)],
                                      wbuf.at[s], sems[s]).wait()

                h_lo, h_hi = _acc_rows(hbuf, H, (s,))
                w_lo, w_hi = _acc_rows(wbuf, W5, (s,))
                u_o[j, pl.ds(0, 16)] = h_lo * (1.0 / H) + w_lo * (1.0 / W5)
                u_o[j, pl.ds(16, 16)] = h_hi * (1.0 / H) + w_hi * (1.0 / W5)

                jn = j + NB

                @pl.when(jn < BW)
                def _():
                    issue(jn, s)
            return 0

        lax.fori_loop(0, BW // NB, row_group, 0)

        for cp in item_cps:
            cp.wait()

        def item_body(j, _):
            for lo, sl in ((0, pl.ds(0, 16)), (16, pl.ds(16, 16))):
                s = bidr[j, sl] + authr[j, sl] + langr[j, sl]
                t01 = tagr[0, j, sl] + tagr[1, j, sl]
                t23 = tagr[2, j, sl] + tagr[3, j, sl]
                tsum = (t01 + t23) + tagr[4, j, sl]
                it_o[j, sl] = s + tsum * (1.0 / T)
            return 0

        lax.fori_loop(0, BW, item_body, 0)

        pltpu.sync_copy(u_o, u_hbm.at[pl.ds(base, BW)])
        pltpu.sync_copy(it_o, item_hbm.at[pl.ds(base, BW)])

    return k(hist3, wish, tags_t, bid, auth, lang,
             book_emb, auth_emb, lang_emb, tag_emb)


_CB = 2048
_BM = 4


def _detile_body(in_ref, out_ref):
    x = in_ref[...]
    eye = jnp.eye(D, dtype=F32)
    q = _BM * _CB // 4
    w = [jnp.transpose(x[:, q * k:q * (k + 1)]) for k in range(4)]
    out_ref[...] = jnp.concatenate(w, axis=1).reshape(-1)


def _detile_table(tbl):
    V = tbl.shape[0]
    bb = _BM * _CB
    grid = (V + bb - 1) // bb
    out = pl.pallas_call(
        _detile_body,
        grid=(grid,),
        in_specs=[pl.BlockSpec((D, bb), lambda c: (0, c))],
        out_specs=pl.BlockSpec((D * bb,), lambda c: (c,)),
        out_shape=jax.ShapeDtypeStruct((grid * bb * D,), F32),
        compiler_params=pltpu.CompilerParams(fuse_transposed_lhs_in_matmul=True),
    )(tbl.T)
    return out.reshape(grid * bb, D)


def _psi(i):
    bb = _BM * _CB
    q = bb // 4
    return (i & ~(bb - 1)) | ((i & (q - 1)) << 2) | ((i & (bb - 1)) // q)


def _tc_towers(u, item, dense8,
               dW1, db1, dW2, db2, dW3, db3,
               uW1, ub1, uW2, ub2, uW3, ub3):
    def body(u_ref, item_ref, dense_ref,
             dW1r, db1r, dW2r, db2r, dW3r, db3r,
             uW1r, ub1r, uW2r, ub2r, uW3r, ub3r, out_ref):
        def mm(x, w):
            return lax.dot_general(x, w, (((1,), (1,)), ((), ())),
                                   preferred_element_type=F32)

        x = jnp.maximum(mm(u_ref[:], uW1r[:]) + ub1r[:], 0.0)
        x = jnp.maximum(mm(x, uW2r[:]) + ub2r[:], 0.0)
        u_emb = mm(x, uW3r[:]) + ub3r[:]

        y = jnp.maximum(mm(dense_ref[:], dW1r[:]) + db1r[:], 0.0)
        y = jnp.maximum(mm(y, dW2r[:]) + db2r[:], 0.0)
        d_e = mm(y, dW3r[:]) + db3r[:]

        i_emb = item_ref[:] + d_e
        out_ref[:] = jnp.sum(u_emb * i_emb, axis=1, keepdims=True)

    return pl.pallas_call(
        body,
        out_shape=jax.ShapeDtypeStruct((B, 1), F32),
    )(u, item, dense8,
      dW1, db1, dW2, db2, dW3, db3,
      uW1, ub1, uW2, ub2, uW3, ub3)


def kernel(hist_ids, wish_ids, bid, auth, lang, tags, dense,
           book_emb, auth_emb, lang_emb, tag_emb,
           dW1, db1, dW2, db2, dW3, db3,
           uW1, ub1, uW2, ub2, uW3, ub3):
    hist3 = _psi(hist_ids).reshape(B, 2, HC)
    tags_t = _psi(tags).T
    u, item = _sc_pool(hist3, _psi(wish_ids), tags_t,
                       _psi(bid), _psi(auth), _psi(lang),
                       _detile_table(book_emb), _detile_table(auth_emb),
                       _detile_table(lang_emb), _detile_table(tag_emb))

    dense8 = jnp.concatenate(
        [dense, jnp.zeros((B, 5), F32)], axis=1)
    dW1p = jnp.concatenate([dW1, jnp.zeros((64, 5), F32)], axis=1)
    return _tc_towers(u, item, dense8,
                      dW1p, db1.reshape(1, 64), dW2, db2.reshape(1, 32),
                      dW3, db3.reshape(1, 32),
                      uW1, ub1.reshape(1, 64), uW2, ub2.reshape(1, 32),
                      uW3, ub3.reshape(1, 32))


# speedup vs baseline: 1.3490x; 1.0123x over previous
import functools

import jax
import jax.numpy as jnp
from jax import lax
from jax.experimental import pallas as pl
from jax.experimental.pallas import tpu as pltpu
from jax.experimental.pallas import tpu_sc as plsc

NC, NS = 2, 16
NW = NC * NS
B = 4096
D = 32
BW = B // NW
H = 200
HC = 100
W5 = 50
T = 5
F32 = jnp.float32
BF16 = jnp.bfloat16


def _acc_rows(buf_ref, n, pre):
    accs = [jnp.zeros((16,), F32) for _ in range(8)]
    for t in range(n):
        c = (t % 4) * 2
        a, b = plsc.unpack(buf_ref[pre + (t,)], format=plsc.PackFormat.INTERLEAVED,
                           preferred_element_type=F32)
        accs[c] = accs[c] + a
        accs[c + 1] = accs[c + 1] + b
    lo = (accs[0] + accs[2]) + (accs[4] + accs[6])
    hi = (accs[1] + accs[3]) + (accs[5] + accs[7])
    return lo, hi


def _sc_pool(hist3, wish, tags_t, bid, auth, lang,
             book_emb, auth_emb, lang_emb, tag_emb):
    mesh = plsc.VectorSubcoreMesh(core_axis_name="c", subcore_axis_name="s",
                                  num_cores=NC, num_subcores=NS)

    @functools.partial(
        pl.kernel,
        out_type=(jax.ShapeDtypeStruct((B, D), F32),
                  jax.ShapeDtypeStruct((B, D), F32)),
        mesh=mesh,
        scratch_types=[
            pltpu.VMEM((BW, 2, HC), jnp.int32),
            pltpu.VMEM((BW, W5), jnp.int32),
            pltpu.VMEM((T, BW), jnp.int32),
            pltpu.VMEM((BW,), jnp.int32),
            pltpu.VMEM((BW,), jnp.int32),
            pltpu.VMEM((BW,), jnp.int32),
            pltpu.VMEM((4, H, D), BF16),
            pltpu.VMEM((4, W5, D), BF16),
            pltpu.VMEM((BW, D), BF16),
            pltpu.VMEM((BW, D), BF16),
            pltpu.VMEM((BW, D), BF16),
            pltpu.VMEM((T, BW, D), BF16),
            pltpu.VMEM((BW, D), F32),
            pltpu.VMEM((BW, D), F32),
            pltpu.SemaphoreType.DMA,
            pltpu.SemaphoreType.DMA,
            pltpu.SemaphoreType.DMA,
            pltpu.SemaphoreType.DMA,
            pltpu.SemaphoreType.DMA,
        ],
        compiler_params=pltpu.CompilerParams(use_tc_tiling_on_sc=False,
                                             needs_layout_passes=False),
    )
    def k(hist_hbm, wish_hbm, tagt_hbm, bid_hbm, auth_hbm, lang_hbm,
          book_hbm, authe_hbm, lange_hbm, tage_hbm,
          u_hbm, item_hbm,
          hist_v, wish_v, tagt_v, bid_v, auth_v, lang_v,
          hbuf, wbuf, bidr, authr, langr, tagr, u_o, it_o,
          sem0, sem1, sem2, sem3, semi):
        wid = lax.axis_index("s") * NC + lax.axis_index("c")
        base = wid * BW
        sems = [sem0, sem1, sem2, sem3]
        NB = 4

        stage_cps = [
            pltpu.async_copy(hist_hbm.at[pl.ds(base, BW)], hist_v, semi),
            pltpu.async_copy(wish_hbm.at[pl.ds(base, BW)], wish_v, semi),
            pltpu.async_copy(tagt_hbm.at[:, pl.ds(base, BW)], tagt_v, semi),
            pltpu.async_copy(bid_hbm.at[pl.ds(base, BW)], bid_v, semi),
            pltpu.async_copy(auth_hbm.at[pl.ds(base, BW)], auth_v, semi),
            pltpu.async_copy(lang_hbm.at[pl.ds(base, BW)], lang_v, semi),
        ]
        for cp in stage_cps:
            cp.wait()

        item_cps = [
            pltpu.async_copy(book_hbm.at[bid_v], bidr, semi),
            pltpu.async_copy(authe_hbm.at[auth_v], authr, semi),
            pltpu.async_copy(lange_hbm.at[lang_v], langr, semi),
        ]
        for t in range(T):
            item_cps.append(
                pltpu.async_copy(tage_hbm.at[tagt_v.at[t]], tagr.at[t], semi))

        def issue(j, s):
            pltpu.async_copy(book_hbm.at[hist_v.at[j, 0]],
                             hbuf.at[s, pl.ds(0, HC)], sems[s])
            pltpu.async_copy(book_hbm.at[hist_v.at[j, 1]],
                             hbuf.at[s, pl.ds(HC, HC)], sems[s])
            pltpu.async_copy(book_hbm.at[wish_v.at[j]], wbuf.at[s], sems[s])

        for s in range(NB):
            issue(s, s)

        def row_group(g, _):
            for s in range(NB):
                j = g * NB + s
                pltpu.make_async_copy(book_hbm.at[pl.ds(0, HC)],
                                      hbuf.at[s, pl.ds(0, HC)], sems[s]).wait()
                pltpu.make_async_copy(book_hbm.at[pl.ds(0, HC)],
                                      hbuf.at[s, pl.ds(HC, HC)], sems[s]).wait()
                pltpu.make_async_copy(book_hbm.at[pl.ds(0, W5)],
                                      wbuf.at[s], sems[s]).wait()

                h_lo, h_hi = _acc_rows(hbuf, H, (s,))
                w_lo, w_hi = _acc_rows(wbuf, W5, (s,))
                u_o[j, pl.ds(0, 16)] = h_lo * (1.0 / H) + w_lo * (1.0 / W5)
                u_o[j, pl.ds(16, 16)] = h_hi * (1.0 / H) + w_hi * (1.0 / W5)

                jn = j + NB

                @pl.when(jn < BW)
                def _():
                    issue(jn, s)
            return 0

        lax.fori_loop(0, BW // NB, row_group, 0)

        for cp in item_cps:
            cp.wait()

        def up(v):
            return plsc.unpack(v, format=plsc.PackFormat.INTERLEAVED,
                               preferred_element_type=F32)

        def item_body(j, _):
            b0, b1 = up(bidr[j])
            a0, a1 = up(authr[j])
            l0, l1 = up(langr[j])
            t0 = [up(tagr[t, j]) for t in range(T)]
            s0 = (b0 + a0) + l0
            s1 = (b1 + a1) + l1
            ts0 = ((t0[0][0] + t0[1][0]) + (t0[2][0] + t0[3][0])) + t0[4][0]
            ts1 = ((t0[0][1] + t0[1][1]) + (t0[2][1] + t0[3][1])) + t0[4][1]
            it_o[j, pl.ds(0, 16)] = s0 + ts0 * (1.0 / T)
            it_o[j, pl.ds(16, 16)] = s1 + ts1 * (1.0 / T)
            return 0

        lax.fori_loop(0, BW, item_body, 0)

        pltpu.sync_copy(u_o, u_hbm.at[pl.ds(base, BW)])
        pltpu.sync_copy(it_o, item_hbm.at[pl.ds(base, BW)])

    return k(hist3, wish, tags_t, bid, auth, lang,
             book_emb, auth_emb, lang_emb, tag_emb)


_CB = 2048
_BM = 8


def _detile_body(in_ref, out_ref):
    x = in_ref[...].astype(BF16)
    q = _BM * _CB // 4
    w = [jnp.transpose(x[:, q * k:q * (k + 1)]) for k in range(4)]
    out_ref[...] = jnp.concatenate(w, axis=1).reshape(-1)


def _detile_table(tbl):
    V = tbl.shape[0]
    bb = _BM * _CB
    grid = (V + bb - 1) // bb
    out = pl.pallas_call(
        _detile_body,
        grid=(grid,),
        in_specs=[pl.BlockSpec((D, bb), lambda c: (0, c))],
        out_specs=pl.BlockSpec((D * bb,), lambda c: (c,)),
        out_shape=jax.ShapeDtypeStruct((grid * bb * D,), BF16),
        compiler_params=pltpu.CompilerParams(fuse_transposed_lhs_in_matmul=True),
    )(tbl.T)
    return out.reshape(grid * bb, D)


def _psi(i):
    bb = _BM * _CB
    q = bb // 4
    return (i & ~(bb - 1)) | ((i & (q - 1)) << 2) | ((i & (bb - 1)) // q)


def _tc_towers(u, item, dense8,
               dW1, db1, dW2, db2, dW3, db3,
               uW1, ub1, uW2, ub2, uW3, ub3):
    def body(u_ref, item_ref, dense_ref,
             dW1r, db1r, dW2r, db2r, dW3r, db3r,
             uW1r, ub1r, uW2r, ub2r, uW3r, ub3r, out_ref):
        def mm(x, w):
            return lax.dot_general(x, w, (((1,), (1,)), ((), ())),
                                   preferred_element_type=F32)

        x = jnp.maximum(mm(u_ref[:], uW1r[:]) + ub1r[:], 0.0)
        x = jnp.maximum(mm(x, uW2r[:]) + ub2r[:], 0.0)
        u_emb = mm(x, uW3r[:]) + ub3r[:]

        y = jnp.maximum(mm(dense_ref[:], dW1r[:]) + db1r[:], 0.0)
        y = jnp.maximum(mm(y, dW2r[:]) + db2r[:], 0.0)
        d_e = mm(y, dW3r[:]) + db3r[:]

        i_emb = item_ref[:] + d_e
        out_ref[:] = jnp.sum(u_emb * i_emb, axis=1, keepdims=True)

    return pl.pallas_call(
        body,
        out_shape=jax.ShapeDtypeStruct((B, 1), F32),
    )(u, item, dense8,
      dW1, db1, dW2, db2, dW3, db3,
      uW1, ub1, uW2, ub2, uW3, ub3)


def kernel(hist_ids, wish_ids, bid, auth, lang, tags, dense,
           book_emb, auth_emb, lang_emb, tag_emb,
           dW1, db1, dW2, db2, dW3, db3,
           uW1, ub1, uW2, ub2, uW3, ub3):
    hist3 = _psi(hist_ids).reshape(B, 2, HC)
    tags_t = _psi(tags).T
    u, item = _sc_pool(hist3, _psi(wish_ids), tags_t,
                       _psi(bid), _psi(auth), _psi(lang),
                       _detile_table(book_emb), _detile_table(auth_emb),
                       _detile_table(lang_emb), _detile_table(tag_emb))

    dense8 = jnp.concatenate(
        [dense, jnp.zeros((B, 5), F32)], axis=1)
    dW1p = jnp.concatenate([dW1, jnp.zeros((64, 5), F32)], axis=1)
    perm = jnp.arange(D).reshape(16, 2).T.reshape(-1)
    return _tc_towers(u, item, dense8,
                      dW1p, db1.reshape(1, 64), dW2, db2.reshape(1, 32),
                      dW3[perm, :], db3[perm].reshape(1, 32),
                      uW1[:, perm], ub1.reshape(1, 64), uW2, ub2.reshape(1, 32),
                      uW3[perm, :], ub3[perm].reshape(1, 32))


# speedup vs baseline: 1.5952x; 1.1825x over previous
import functools

import jax
import jax.numpy as jnp
from jax import lax
from jax.experimental import pallas as pl
from jax.experimental.pallas import tpu as pltpu
from jax.experimental.pallas import tpu_sc as plsc

NC, NS = 2, 16
NW = NC * NS
B = 4096
D = 32
DW = D // 2
BW = B // NW
H = 200
HC = 100
W5 = 50
T = 5
F32 = jnp.float32
BF16 = jnp.bfloat16


def _unpack_row(v):
    vb = plsc.bitcast(v, BF16)
    return plsc.unpack(vb, format=plsc.PackFormat.INTERLEAVED,
                       preferred_element_type=F32)


def _acc_rows(buf_ref, n, pre):
    accs = [jnp.zeros((16,), F32) for _ in range(8)]
    for t in range(n):
        c = (t % 4) * 2
        a, b = _unpack_row(buf_ref[pre + (t,)])
        accs[c] = accs[c] + a
        accs[c + 1] = accs[c + 1] + b
    lo = (accs[0] + accs[2]) + (accs[4] + accs[6])
    hi = (accs[1] + accs[3]) + (accs[5] + accs[7])
    return lo, hi


def _sc_pool(hist3, wish, tags_t, bid, auth, lang,
             book_emb, auth_emb, lang_emb, tag_emb):
    mesh = plsc.VectorSubcoreMesh(core_axis_name="c", subcore_axis_name="s",
                                  num_cores=NC, num_subcores=NS)

    @functools.partial(
        pl.kernel,
        out_type=(jax.ShapeDtypeStruct((B, D), F32),
                  jax.ShapeDtypeStruct((B, D), F32)),
        mesh=mesh,
        scratch_types=[
            pltpu.VMEM((BW, 2, HC), jnp.int32),
            pltpu.VMEM((BW, W5), jnp.int32),
            pltpu.VMEM((T, BW), jnp.int32),
            pltpu.VMEM((BW,), jnp.int32),
            pltpu.VMEM((BW,), jnp.int32),
            pltpu.VMEM((BW,), jnp.int32),
            pltpu.VMEM((4, H, DW), jnp.int32),
            pltpu.VMEM((4, W5, DW), jnp.int32),
            pltpu.VMEM((BW, DW), jnp.int32),
            pltpu.VMEM((BW, DW), jnp.int32),
            pltpu.VMEM((BW, DW), jnp.int32),
            pltpu.VMEM((T, BW, DW), jnp.int32),
            pltpu.VMEM((BW, D), F32),
            pltpu.VMEM((BW, D), F32),
            pltpu.SemaphoreType.DMA,
            pltpu.SemaphoreType.DMA,
            pltpu.SemaphoreType.DMA,
            pltpu.SemaphoreType.DMA,
            pltpu.SemaphoreType.DMA,
        ],
        compiler_params=pltpu.CompilerParams(use_tc_tiling_on_sc=False,
                                             needs_layout_passes=False),
    )
    def k(hist_hbm, wish_hbm, tagt_hbm, bid_hbm, auth_hbm, lang_hbm,
          book_hbm, authe_hbm, lange_hbm, tage_hbm,
          u_hbm, item_hbm,
          hist_v, wish_v, tagt_v, bid_v, auth_v, lang_v,
          hbuf, wbuf, bidr, authr, langr, tagr, u_o, it_o,
          sem0, sem1, sem2, sem3, semi):
        wid = lax.axis_index("s") * NC + lax.axis_index("c")
        base = wid * BW
        sems = [sem0, sem1, sem2, sem3]
        NB = 4

        stage_cps = [
            pltpu.async_copy(hist_hbm.at[pl.ds(base, BW)], hist_v, semi),
            pltpu.async_copy(wish_hbm.at[pl.ds(base, BW)], wish_v, semi),
            pltpu.async_copy(tagt_hbm.at[:, pl.ds(base, BW)], tagt_v, semi),
            pltpu.async_copy(bid_hbm.at[pl.ds(base, BW)], bid_v, semi),
            pltpu.async_copy(auth_hbm.at[pl.ds(base, BW)], auth_v, semi),
            pltpu.async_copy(lang_hbm.at[pl.ds(base, BW)], lang_v, semi),
        ]
        for cp in stage_cps:
            cp.wait()

        item_cps = [
            pltpu.async_copy(book_hbm.at[bid_v], bidr, semi),
            pltpu.async_copy(authe_hbm.at[auth_v], authr, semi),
            pltpu.async_copy(lange_hbm.at[lang_v], langr, semi),
        ]
        for t in range(T):
            item_cps.append(
                pltpu.async_copy(tage_hbm.at[tagt_v.at[t]], tagr.at[t], semi))

        def issue(j, s):
            pltpu.async_copy(book_hbm.at[hist_v.at[j, 0]],
                             hbuf.at[s, pl.ds(0, HC)], sems[s])
            pltpu.async_copy(book_hbm.at[hist_v.at[j, 1]],
                             hbuf.at[s, pl.ds(HC, HC)], sems[s])
            pltpu.async_copy(book_hbm.at[wish_v.at[j]], wbuf.at[s], sems[s])

        for s in range(NB):
            issue(s, s)

        def row_group(g, _):
            for s in range(NB):
                j = g * NB + s
                pltpu.make_async_copy(book_hbm.at[pl.ds(0, HC)],
                                      hbuf.at[s, pl.ds(0, HC)], sems[s]).wait()
                pltpu.make_async_copy(book_hbm.at[pl.ds(0, HC)],
                                      hbuf.at[s, pl.ds(HC, HC)], sems[s]).wait()
                pltpu.make_async_copy(book_hbm.at[pl.ds(0, W5)],
                                      wbuf.at[s], sems[s]).wait()

                h_lo, h_hi = _acc_rows(hbuf, H, (s,))
                w_lo, w_hi = _acc_rows(wbuf, W5, (s,))
                u_o[j, pl.ds(0, 16)] = h_lo * (1.0 / H) + w_lo * (1.0 / W5)
                u_o[j, pl.ds(16, 16)] = h_hi * (1.0 / H) + w_hi * (1.0 / W5)

                jn = j + NB

                @pl.when(jn < BW)
                def _():
                    issue(jn, s)
            return 0

        lax.fori_loop(0, BW // NB, row_group, 0)

        for cp in item_cps:
            cp.wait()

        def item_body(j, _):
            b0, b1 = _unpack_row(bidr[j])
            a0, a1 = _unpack_row(authr[j])
            l0, l1 = _unpack_row(langr[j])
            tg = [_unpack_row(tagr[t, j]) for t in range(T)]
            s0 = (b0 + a0) + l0
            s1 = (b1 + a1) + l1
            ts0 = ((tg[0][0] + tg[1][0]) + (tg[2][0] + tg[3][0])) + tg[4][0]
            ts1 = ((tg[0][1] + tg[1][1]) + (tg[2][1] + tg[3][1])) + tg[4][1]
            it_o[j, pl.ds(0, 16)] = s0 + ts0 * (1.0 / T)
            it_o[j, pl.ds(16, 16)] = s1 + ts1 * (1.0 / T)
            return 0

        lax.fori_loop(0, BW, item_body, 0)

        pltpu.sync_copy(u_o, u_hbm.at[pl.ds(base, BW)])
        pltpu.sync_copy(it_o, item_hbm.at[pl.ds(base, BW)])

    return k(hist3, wish, tags_t, bid, auth, lang,
             book_emb, auth_emb, lang_emb, tag_emb)


_BB = 16384
_NCK = 8
_Q = _BB // _NCK


def _rne_bf16_bits(u):
    return (u + 0x7FFF + ((u >> 16) & 1)) >> 16


def _detile_body(in_ref, out_ref):
    x = lax.bitcast_convert_type(in_ref[...], jnp.uint32)
    lo = _rne_bf16_bits(x[:16, :])
    hi = _rne_bf16_bits(x[16:, :])
    w = lax.bitcast_convert_type(lo | (hi << 16), jnp.int32)
    y = jnp.concatenate(
        [jnp.transpose(w[:, _Q * k:_Q * (k + 1)]) for k in range(_NCK)],
        axis=1)
    out_ref[...] = y.reshape(-1)


def _detile_table(tbl):
    V = tbl.shape[0]
    grid = (V + _BB - 1) // _BB
    out = pl.pallas_call(
        _detile_body,
        grid=(grid,),
        in_specs=[pl.BlockSpec((D, _BB), lambda c: (0, c))],
        out_specs=pl.BlockSpec((DW * _BB,), lambda c: (c,)),
        out_shape=jax.ShapeDtypeStruct((grid * _BB * DW,), jnp.int32),
    )(tbl.T)
    return out.reshape(grid * _BB, DW)


def _psi(i):
    return (i & ~(_BB - 1)) | ((i & (_Q - 1)) * _NCK) | ((i & (_BB - 1)) // _Q)


def _tc_towers(u, item, dense8,
               dW1, db1, dW2, db2, dW3, db3,
               uW1, ub1, uW2, ub2, uW3, ub3):
    def body(u_ref, item_ref, dense_ref,
             dW1r, db1r, dW2r, db2r, dW3r, db3r,
             uW1r, ub1r, uW2r, ub2r, uW3r, ub3r, out_ref):
        def mm(x, w):
            return lax.dot_general(x, w, (((1,), (1,)), ((), ())),
                                   preferred_element_type=F32)

        x = jnp.maximum(mm(u_ref[:], uW1r[:]) + ub1r[:], 0.0)
        x = jnp.maximum(mm(x, uW2r[:]) + ub2r[:], 0.0)
        u_emb = mm(x, uW3r[:]) + ub3r[:]

        y = jnp.maximum(mm(dense_ref[:], dW1r[:]) + db1r[:], 0.0)
        y = jnp.maximum(mm(y, dW2r[:]) + db2r[:], 0.0)
        d_e = mm(y, dW3r[:]) + db3r[:]

        i_emb = item_ref[:] + d_e
        out_ref[:] = jnp.sum(u_emb * i_emb, axis=1, keepdims=True)

    return pl.pallas_call(
        body,
        out_shape=jax.ShapeDtypeStruct((B, 1), F32),
    )(u, item, dense8,
      dW1, db1, dW2, db2, dW3, db3,
      uW1, ub1, uW2, ub2, uW3, ub3)


def kernel(hist_ids, wish_ids, bid, auth, lang, tags, dense,
           book_emb, auth_emb, lang_emb, tag_emb,
           dW1, db1, dW2, db2, dW3, db3,
           uW1, ub1, uW2, ub2, uW3, ub3):
    hist3 = _psi(hist_ids).reshape(B, 2, HC)
    tags_t = _psi(tags).T
    u, item = _sc_pool(hist3, _psi(wish_ids), tags_t,
                       _psi(bid), _psi(auth), _psi(lang),
                       _detile_table(book_emb), _detile_table(auth_emb),
                       _detile_table(lang_emb), _detile_table(tag_emb))

    dense8 = jnp.concatenate(
        [dense, jnp.zeros((B, 5), F32)], axis=1)
    dW1p = jnp.concatenate([dW1, jnp.zeros((64, 5), F32)], axis=1)
    return _tc_towers(u, item, dense8,
                      dW1p, db1.reshape(1, 64), dW2, db2.reshape(1, 32),
                      dW3, db3.reshape(1, 32),
                      uW1, ub1.reshape(1, 64), uW2, ub2.reshape(1, 32),
                      uW3, ub3.reshape(1, 32))


# speedup vs baseline: 1.5990x; 1.0024x over previous
import functools

import jax
import jax.numpy as jnp
from jax import lax
from jax.experimental import pallas as pl
from jax.experimental.pallas import tpu as pltpu
from jax.experimental.pallas import tpu_sc as plsc

NC, NS = 2, 16
NW = NC * NS
B = 4096
D = 32
DW = D // 2
BW = B // NW
H = 200
HC = 100
W5 = 50
T = 5
F32 = jnp.float32
BF16 = jnp.bfloat16


def _unpack_row(v):
    vb = plsc.bitcast(v, BF16)
    return plsc.unpack(vb, format=plsc.PackFormat.INTERLEAVED,
                       preferred_element_type=F32)


def _acc_rows(buf_ref, n, pre):
    accs = [jnp.zeros((16,), F32) for _ in range(8)]
    for t in range(n):
        c = (t % 4) * 2
        a, b = _unpack_row(buf_ref[pre + (t,)])
        accs[c] = accs[c] + a
        accs[c + 1] = accs[c + 1] + b
    lo = (accs[0] + accs[2]) + (accs[4] + accs[6])
    hi = (accs[1] + accs[3]) + (accs[5] + accs[7])
    return lo, hi


def _sc_pool(hist3, wish, tags_t, bid, auth, lang,
             book_emb, auth_emb, lang_emb, tag_emb):
    mesh = plsc.VectorSubcoreMesh(core_axis_name="c", subcore_axis_name="s",
                                  num_cores=NC, num_subcores=NS)

    @functools.partial(
        pl.kernel,
        out_type=(jax.ShapeDtypeStruct((B, D), F32),
                  jax.ShapeDtypeStruct((B, D), F32)),
        mesh=mesh,
        scratch_types=[
            pltpu.VMEM((BW, 2, HC), jnp.int32),
            pltpu.VMEM((BW, W5), jnp.int32),
            pltpu.VMEM((T, BW), jnp.int32),
            pltpu.VMEM((BW,), jnp.int32),
            pltpu.VMEM((BW,), jnp.int32),
            pltpu.VMEM((BW,), jnp.int32),
            pltpu.VMEM((4, H, DW), jnp.int32),
            pltpu.VMEM((4, W5, DW), jnp.int32),
            pltpu.VMEM((BW, DW), jnp.int32),
            pltpu.VMEM((BW, DW), jnp.int32),
            pltpu.VMEM((BW, DW), jnp.int32),
            pltpu.VMEM((T, BW, DW), jnp.int32),
            pltpu.VMEM((BW, D), F32),
            pltpu.VMEM((BW, D), F32),
            pltpu.SemaphoreType.DMA,
            pltpu.SemaphoreType.DMA,
            pltpu.SemaphoreType.DMA,
            pltpu.SemaphoreType.DMA,
            pltpu.SemaphoreType.DMA,
        ],
        compiler_params=pltpu.CompilerParams(use_tc_tiling_on_sc=False,
                                             needs_layout_passes=False),
    )
    def k(hist_hbm, wish_hbm, tagt_hbm, bid_hbm, auth_hbm, lang_hbm,
          book_hbm, authe_hbm, lange_hbm, tage_hbm,
          u_hbm, item_hbm,
          hist_v, wish_v, tagt_v, bid_v, auth_v, lang_v,
          hbuf, wbuf, bidr, authr, langr, tagr, u_o, it_o,
          sem0, sem1, sem2, sem3, semi):
        wid = lax.axis_index("s") * NC + lax.axis_index("c")
        base = wid * BW
        sems = [sem0, sem1, sem2, sem3]
        NB = 4

        stage_cps = [
            pltpu.async_copy(hist_hbm.at[pl.ds(base, BW)], hist_v, semi),
            pltpu.async_copy(wish_hbm.at[pl.ds(base, BW)], wish_v, semi),
            pltpu.async_copy(tagt_hbm.at[:, pl.ds(base, BW)], tagt_v, semi),
            pltpu.async_copy(bid_hbm.at[pl.ds(base, BW)], bid_v, semi),
            pltpu.async_copy(auth_hbm.at[pl.ds(base, BW)], auth_v, semi),
            pltpu.async_copy(lang_hbm.at[pl.ds(base, BW)], lang_v, semi),
        ]
        for cp in stage_cps:
            cp.wait()

        item_cps = [
            pltpu.async_copy(book_hbm.at[bid_v], bidr, semi),
            pltpu.async_copy(authe_hbm.at[auth_v], authr, semi),
            pltpu.async_copy(lange_hbm.at[lang_v], langr, semi),
        ]
        for t in range(T):
            item_cps.append(
                pltpu.async_copy(tage_hbm.at[tagt_v.at[t]], tagr.at[t], semi))

        def issue(j, s):
            pltpu.async_copy(book_hbm.at[hist_v.at[j, 0]],
                             hbuf.at[s, pl.ds(0, HC)], sems[s])
            pltpu.async_copy(book_hbm.at[hist_v.at[j, 1]],
                             hbuf.at[s, pl.ds(HC, HC)], sems[s])
            pltpu.async_copy(book_hbm.at[wish_v.at[j]], wbuf.at[s], sems[s])

        for s in range(NB):
            issue(s, s)

        def row_group(g, _):
            for s in range(NB):
                j = g * NB + s
                pltpu.make_async_copy(book_hbm.at[pl.ds(0, HC)],
                                      hbuf.at[s, pl.ds(0, HC)], sems[s]).wait()
                pltpu.make_async_copy(book_hbm.at[pl.ds(0, HC)],
                                      hbuf.at[s, pl.ds(HC, HC)], sems[s]).wait()
                pltpu.make_async_copy(book_hbm.at[pl.ds(0, W5)],
                                      wbuf.at[s], sems[s]).wait()

                h_lo, h_hi = _acc_rows(hbuf, H, (s,))
                w_lo, w_hi = _acc_rows(wbuf, W5, (s,))
                u_o[j, pl.ds(0, 16)] = h_lo * (1.0 / H) + w_lo * (1.0 / W5)
                u_o[j, pl.ds(16, 16)] = h_hi * (1.0 / H) + w_hi * (1.0 / W5)

                jn = j + NB

                @pl.when(jn < BW)
                def _():
                    issue(jn, s)
            return 0

        lax.fori_loop(0, BW // NB, row_group, 0)

        for cp in item_cps:
            cp.wait()

        def item_body(j, _):
            b0, b1 = _unpack_row(bidr[j])
            a0, a1 = _unpack_row(authr[j])
            l0, l1 = _unpack_row(langr[j])
            tg = [_unpack_row(tagr[t, j]) for t in range(T)]
            s0 = (b0 + a0) + l0
            s1 = (b1 + a1) + l1
            ts0 = ((tg[0][0] + tg[1][0]) + (tg[2][0] + tg[3][0])) + tg[4][0]
            ts1 = ((tg[0][1] + tg[1][1]) + (tg[2][1] + tg[3][1])) + tg[4][1]
            it_o[j, pl.ds(0, 16)] = s0 + ts0 * (1.0 / T)
            it_o[j, pl.ds(16, 16)] = s1 + ts1 * (1.0 / T)
            return 0

        lax.fori_loop(0, BW, item_body, 0)

        pltpu.sync_copy(u_o, u_hbm.at[pl.ds(base, BW)])
        pltpu.sync_copy(it_o, item_hbm.at[pl.ds(base, BW)])

    return k(hist3, wish, tags_t, bid, auth, lang,
             book_emb, auth_emb, lang_emb, tag_emb)


_BB = 8192
_NCK = 8
_Q = _BB // _NCK


def _rne_bf16_bits(u):
    return (u + 0x7FFF + ((u >> 16) & 1)) >> 16


def _detile_body(in_ref, out_ref):
    x = lax.bitcast_convert_type(in_ref[...], jnp.uint32)
    lo = _rne_bf16_bits(x[:16, :])
    hi = _rne_bf16_bits(x[16:, :])
    w = lax.bitcast_convert_type(lo | (hi << 16), jnp.int32)
    y = jnp.concatenate(
        [jnp.transpose(w[:, _Q * k:_Q * (k + 1)]) for k in range(_NCK)],
        axis=1)
    out_ref[...] = y.reshape(-1)


def _detile_table(tbl):
    V = tbl.shape[0]
    grid = (V + _BB - 1) // _BB
    out = pl.pallas_call(
        _detile_body,
        grid=(grid,),
        in_specs=[pl.BlockSpec((D, _BB), lambda c: (0, c))],
        out_specs=pl.BlockSpec((DW * _BB,), lambda c: (c,)),
        out_shape=jax.ShapeDtypeStruct((grid * _BB * DW,), jnp.int32),
    )(tbl.T)
    return out.reshape(grid * _BB, DW)


def _psi(i):
    return (i & ~(_BB - 1)) | ((i & (_Q - 1)) * _NCK) | ((i & (_BB - 1)) // _Q)


def _tc_towers(u, item, dense8,
               dW1, db1, dW2, db2, dW3, db3,
               uW1, ub1, uW2, ub2, uW3, ub3):
    def body(u_ref, item_ref, dense_ref,
             dW1r, db1r, dW2r, db2r, dW3r, db3r,
             uW1r, ub1r, uW2r, ub2r, uW3r, ub3r, out_ref):
        def mm(x, w):
            return lax.dot_general(x, w, (((1,), (1,)), ((), ())),
                                   preferred_element_type=F32)

        x = jnp.maximum(mm(u_ref[:], uW1r[:]) + ub1r[:], 0.0)
        x = jnp.maximum(mm(x, uW2r[:]) + ub2r[:], 0.0)
        u_emb = mm(x, uW3r[:]) + ub3r[:]

        y = jnp.maximum(mm(dense_ref[:], dW1r[:]) + db1r[:], 0.0)
        y = jnp.maximum(mm(y, dW2r[:]) + db2r[:], 0.0)
        d_e = mm(y, dW3r[:]) + db3r[:]

        i_emb = item_ref[:] + d_e
        out_ref[:] = jnp.sum(u_emb * i_emb, axis=1, keepdims=True)

    return pl.pallas_call(
        body,
        out_shape=jax.ShapeDtypeStruct((B, 1), F32),
    )(u, item, dense8,
      dW1, db1, dW2, db2, dW3, db3,
      uW1, ub1, uW2, ub2, uW3, ub3)


def kernel(hist_ids, wish_ids, bid, auth, lang, tags, dense,
           book_emb, auth_emb, lang_emb, tag_emb,
           dW1, db1, dW2, db2, dW3, db3,
           uW1, ub1, uW2, ub2, uW3, ub3):
    hist3 = _psi(hist_ids).reshape(B, 2, HC)
    tags_t = _psi(tags).T
    u, item = _sc_pool(hist3, _psi(wish_ids), tags_t,
                       _psi(bid), _psi(auth), _psi(lang),
                       _detile_table(book_emb), _detile_table(auth_emb),
                       _detile_table(lang_emb), _detile_table(tag_emb))

    dense8 = jnp.concatenate(
        [dense, jnp.zeros((B, 5), F32)], axis=1)
    dW1p = jnp.concatenate([dW1, jnp.zeros((64, 5), F32)], axis=1)
    return _tc_towers(u, item, dense8,
                      dW1p, db1.reshape(1, 64), dW2, db2.reshape(1, 32),
                      dW3, db3.reshape(1, 32),
                      uW1, ub1.reshape(1, 64), uW2, ub2.reshape(1, 32),
                      uW3, ub3.reshape(1, 32))
